# baseline (speedup 1.0000x reference)
import functools

import jax
import jax.numpy as jnp
from jax import lax
from jax.experimental import pallas as pl
from jax.experimental.pallas import tpu as pltpu
from jax.experimental.pallas import tpu_sc as plsc

_B = 16384
_D = 64
_E = 100000
_L = 16
_NC = 2
_NS = 16
_NW = _NC * _NS
_PW = _B // _NW
_S = 4096
_NSTR = _B // _S
_CCH = 128


_H0 = 50048
_H1 = _E - _H0


def _phase1_body(headT, tailT, relT, rinvT,
                 h_idx, r_idx, t_idx,
                 a_out, b_out, c_out, d_out, e_out, f_out,
                 half0_v, half1_v, val_v, idx0_v, idx1_v,
                 sem_i, sem_o, sem_p0, sem_p1):
  wid = lax.axis_index("s") * _NC + lax.axis_index("c")
  idx_bufs = (idx0_v, idx1_v)

  tbls = (headT, headT, tailT, tailT, relT, relT, rinvT, rinvT)
  roles_per_rnd = (
      ((0, 0), (2, 3)), ((0, 0), (2, 3)),
      ((2, 2), (0, 5)), ((2, 2), (0, 5)),
      ((1, 1),), ((1, 1),),
      ((1, 4),), ((1, 4),),
  )
  idx_list = (h_idx, r_idx, t_idx)
  out_list = (a_out, b_out, c_out, d_out, e_out, f_out)

  def feat(rnd):
    return (rnd % 2) * 32 + wid

  def prefetch(rnd, half):
    if rnd >= 8:
      return
    t = tbls[rnd]
    if half == 0:
      pltpu.async_copy(t.at[feat(rnd), pl.ds(0, _H0)], half0_v, sem_p0)
    else:
      pltpu.async_copy(t.at[feat(rnd), pl.ds(_H0, _H1)], half1_v, sem_p1)

  def gather_pass(idx_hbm, is_b):
    for s in range(_NSTR):
      idx_v = idx_bufs[s % 2]
      pltpu.make_async_copy(idx_hbm.at[pl.ds(s * _S, _S)], idx_v,
                            sem_i).wait()
      if s + 1 < _NSTR:
        pltpu.async_copy(idx_hbm.at[pl.ds((s + 1) * _S, _S)],
                         idx_bufs[(s + 1) % 2], sem_i)

      def gbody(g, carry):
        for u in range(4):
          o = (g * 4 + u) * _L
          sl = pl.ds(o, _L)
          vsl = pl.ds(s * _S + o, _L)
          idx = idx_v[sl]
          if not is_b:
            loc = jnp.minimum(idx, _H0 - 1)
            val_v[vsl] = plsc.load_gather(half0_v, [loc])
          else:
            loc = jnp.maximum(idx - _H0, 0)
            g1 = plsc.load_gather(half1_v, [loc])
            val_v[vsl] = jnp.where(idx >= _H0, g1, val_v[vsl])
        return carry

      lax.fori_loop(0, _S // (4 * _L), gbody, 0)

  prefetch(0, 0)
  prefetch(0, 1)
  out_cp = None
  for rnd in range(8):
    d = feat(rnd)
    roles = roles_per_rnd[rnd]
    nr = len(roles)
    for ri, (ii, oo) in enumerate(roles):
      idx_hbm = idx_list[ii]
      out_hbm = out_list[oo]
      pltpu.async_copy(idx_hbm.at[pl.ds(0, _S)], idx_bufs[0], sem_i)
      if ri == 0:
        pltpu.make_async_copy(
            tbls[rnd].at[feat(rnd), pl.ds(0, _H0)], half0_v, sem_p0).wait()
      if out_cp is not None:
        out_cp.wait()
        out_cp = None
      gather_pass(idx_hbm, False)
      if ri == nr - 1:
        prefetch(rnd + 1, 0)
      pltpu.async_copy(idx_hbm.at[pl.ds(0, _S)], idx_bufs[0], sem_i)
      if ri == 0:
        pltpu.make_async_copy(
            tbls[rnd].at[feat(rnd), pl.ds(_H0, _H1)], half1_v,
            sem_p1).wait()
      gather_pass(idx_hbm, True)
      if ri == nr - 1:
        prefetch(rnd + 1, 1)
      out_cp = pltpu.async_copy(val_v, out_hbm.at[d], sem_o)
  out_cp.wait()


def _phase2_tc(a_ref, b_ref, c_ref, d_ref, e_ref, f_ref, out_ref):
  prod = a_ref[...] * b_ref[...] * c_ref[...] \
      + d_ref[...] * e_ref[...] * f_ref[...]
  out_ref[...] = 0.5 * jnp.sum(prod, axis=0)


@jax.jit
def _simple_score(h_idx, r_idx, t_idx, headT, tailT, relT, rinvT):
  mesh = plsc.VectorSubcoreMesh(
      core_axis_name="c", subcore_axis_name="s",
      num_cores=_NC, num_subcores=_NS)
  gmat = jax.ShapeDtypeStruct((_D, _B), jnp.float32)
  params = pltpu.CompilerParams(needs_layout_passes=False)
  p1 = functools.partial(
      pl.kernel,
      out_type=(gmat,) * 6,
      mesh=mesh,
      compiler_params=params,
      scratch_types=[
          pltpu.VMEM((_H0,), jnp.float32),
          pltpu.VMEM((_H1,), jnp.float32),
          pltpu.VMEM((_B,), jnp.float32),
          pltpu.VMEM((_S,), jnp.int32),
          pltpu.VMEM((_S,), jnp.int32),
          pltpu.SemaphoreType.DMA,
          pltpu.SemaphoreType.DMA,
          pltpu.SemaphoreType.DMA,
          pltpu.SemaphoreType.DMA,
      ],
  )(_phase1_body)
  a, b, c, d, e, f = p1(headT, tailT, relT, rinvT, h_idx, r_idx, t_idx)

  blk = 2048
  in_spec = pl.BlockSpec((_D, blk), lambda i: (0, i))
  p2 = pl.pallas_call(
      _phase2_tc,
      out_shape=jax.ShapeDtypeStruct((_B,), jnp.float32),
      grid=(_B // blk,),
      in_specs=[in_spec] * 6,
      out_specs=pl.BlockSpec((blk,), lambda i: (i,)),
  )
  return p2(a, b, c, d, e, f)


def kernel(sample, head_embedding, tail_embedding, relation_embedding,
           relation_inverse_embedding):
  sample = sample.astype(jnp.int32)
  h_idx = sample[:, 0]
  r_idx = sample[:, 1]
  t_idx = sample[:, 2]
  return _simple_score(h_idx, r_idx, t_idx,
                       head_embedding.T, tail_embedding.T,
                       relation_embedding.T, relation_inverse_embedding.T)


# speedup vs baseline: 2.1588x; 2.1588x over previous
import functools

import jax
import jax.numpy as jnp
from jax import lax
from jax.experimental import pallas as pl
from jax.experimental.pallas import tpu as pltpu
from jax.experimental.pallas import tpu_sc as plsc

_B = 16384
_D = 64
_E = 100000
_L = 16
_NC = 2
_NS = 16
_NW = _NC * _NS
_PW = _B // _NW
_S = 4096
_NSTR = _B // _S
_CCH = 128


def _phase1_body(headT, tailT, relT, rinvT,
                 h_idx, r_idx, t_idx,
                 a_out, b_out, c_out, d_out, e_out, f_out,
                 plane_v, idx0_v, idx1_v, val0_v, val1_v,
                 sem_i, sem_o):
  wid = lax.axis_index("s") * _NC + lax.axis_index("c")
  idx_bufs = (idx0_v, idx1_v)
  val_bufs = (val0_v, val1_v)

  def gather_role(d, idx_hbm, out_hbm):
    pltpu.async_copy(idx_hbm.at[pl.ds(0, _S)], idx_bufs[0], sem_i)
    out_cps = []
    for s in range(_NSTR):
      idx_v = idx_bufs[s % 2]
      val_v = val_bufs[s % 2]
      pltpu.make_async_copy(idx_hbm.at[pl.ds(s * _S, _S)], idx_v,
                            sem_i).wait()
      if s + 1 < _NSTR:
        pltpu.async_copy(idx_hbm.at[pl.ds((s + 1) * _S, _S)],
                         idx_bufs[(s + 1) % 2], sem_i)
      if s >= 2:
        out_cps[s - 2].wait()

      def gbody(g, carry):
        for u in range(16):
          sl = pl.ds((g * 16 + u) * _L, _L)
          val_v[sl] = plsc.load_gather(plane_v, [idx_v[sl]])
        return carry

      lax.fori_loop(0, _S // (16 * _L), gbody, 0)
      out_cps.append(
          pltpu.async_copy(val_v, out_hbm.at[d, pl.ds(s * _S, _S)], sem_o))
    for c in out_cps[max(0, _NSTR - 2):]:
      c.wait()

  for rnd in range(8):
    tbl = (headT, headT, tailT, tailT, relT, relT, rinvT, rinvT)[rnd]
    d = (rnd % 2) * 32 + wid
    pltpu.sync_copy(tbl.at[d], plane_v)
    if rnd < 2:
      gather_role(d, h_idx, a_out)
      gather_role(d, t_idx, d_out)
    elif rnd < 4:
      gather_role(d, t_idx, c_out)
      gather_role(d, h_idx, f_out)
    elif rnd < 6:
      gather_role(d, r_idx, b_out)
    else:
      gather_role(d, r_idx, e_out)


def _phase2_tc(a_ref, b_ref, c_ref, d_ref, e_ref, f_ref, out_ref):
  prod = a_ref[...] * b_ref[...] * c_ref[...] \
      + d_ref[...] * e_ref[...] * f_ref[...]
  out_ref[...] = 0.5 * jnp.sum(prod, axis=0)


@jax.jit
def _simple_score(h_idx, r_idx, t_idx, headT, tailT, relT, rinvT):
  mesh = plsc.VectorSubcoreMesh(
      core_axis_name="c", subcore_axis_name="s",
      num_cores=_NC, num_subcores=_NS)
  gmat = jax.ShapeDtypeStruct((_D, _B), jnp.float32)
  params = pltpu.CompilerParams(needs_layout_passes=False)
  p1 = functools.partial(
      pl.kernel,
      out_type=(gmat,) * 6,
      mesh=mesh,
      compiler_params=params,
      scratch_types=[
          pltpu.VMEM((_E,), jnp.float32),
          pltpu.VMEM((_S,), jnp.int32),
          pltpu.VMEM((_S,), jnp.int32),
          pltpu.VMEM((_S,), jnp.float32),
          pltpu.VMEM((_S,), jnp.float32),
          pltpu.SemaphoreType.DMA,
          pltpu.SemaphoreType.DMA,
      ],
  )(_phase1_body)
  a, b, c, d, e, f = p1(headT, tailT, relT, rinvT, h_idx, r_idx, t_idx)

  blk = 2048
  in_spec = pl.BlockSpec((_D, blk), lambda i: (0, i))
  p2 = pl.pallas_call(
      _phase2_tc,
      out_shape=jax.ShapeDtypeStruct((_B,), jnp.float32),
      grid=(_B // blk,),
      in_specs=[in_spec] * 6,
      out_specs=pl.BlockSpec((blk,), lambda i: (i,)),
  )
  return p2(a, b, c, d, e, f)


def kernel(sample, head_embedding, tail_embedding, relation_embedding,
           relation_inverse_embedding):
  sample = sample.astype(jnp.int32)
  h_idx = sample[:, 0]
  r_idx = sample[:, 1]
  t_idx = sample[:, 2]
  return _simple_score(h_idx, r_idx, t_idx,
                       head_embedding.T, tail_embedding.T,
                       relation_embedding.T, relation_inverse_embedding.T)


# speedup vs baseline: 2.3160x; 1.0728x over previous
import functools

import jax
import jax.numpy as jnp
from jax import lax
from jax.experimental import pallas as pl
from jax.experimental.pallas import tpu as pltpu
from jax.experimental.pallas import tpu_sc as plsc

_B = 16384
_D = 64
_E = 100000
_L = 16
_NC = 2
_NS = 16
_NW = _NC * _NS
_PW = _B // _NW
_S = 4096
_NSTR = _B // _S
_CCH = 128


def _phase1_body(headT, tailT, relT, rinvT,
                 h_idx, r_idx, t_idx,
                 a_out, b_out, c_out, d_out, e_out, f_out,
                 plane_v, h_sp, r_sp, t_sp, idx0_v, idx1_v, val0_v, val1_v,
                 sem_i, sem_o, sem_p, sem_s):
  wid = lax.axis_index("s") * _NC + lax.axis_index("c")
  sid = lax.axis_index("s")
  idx_bufs = (idx0_v, idx1_v)
  val_bufs = (val0_v, val1_v)

  @pl.when(sid == 0)
  def _stage_idx():
    pltpu.sync_copy(h_idx, h_sp)
    pltpu.sync_copy(r_idx, r_sp)
    pltpu.sync_copy(t_idx, t_sp)
  plsc.subcore_barrier()
  idx_srcs = (h_sp, r_sp, t_sp)

  def gather_role(d, idx_row, out_hbm):
    idx_src = idx_srcs[idx_row]
    pltpu.async_copy(idx_src.at[pl.ds(0, _S)], idx_bufs[0], sem_i)
    out_cps = []
    for s in range(_NSTR):
      idx_v = idx_bufs[s % 2]
      val_v = val_bufs[s % 2]
      pltpu.make_async_copy(idx_src.at[pl.ds(s * _S, _S)], idx_v,
                            sem_i).wait()
      if s + 1 < _NSTR:
        pltpu.async_copy(idx_src.at[pl.ds((s + 1) * _S, _S)],
                         idx_bufs[(s + 1) % 2], sem_i)
      if s >= 2:
        out_cps[s - 2].wait()

      def gbody(g, carry):
        for u in range(16):
          sl = pl.ds((g * 16 + u) * _L, _L)
          val_v[sl] = plsc.load_gather(plane_v, [idx_v[sl]])
        return carry

      lax.fori_loop(0, _S // (16 * _L), gbody, 0)
      out_cps.append(
          pltpu.async_copy(val_v, out_hbm.at[d, pl.ds(s * _S, _S)], sem_o))
    for c in out_cps[max(0, _NSTR - 2):]:
      c.wait()

  for rnd in range(8):
    tbl = (headT, headT, tailT, tailT, relT, relT, rinvT, rinvT)[rnd]
    d = (rnd % 2) * 32 + wid
    pltpu.sync_copy(tbl.at[d], plane_v)
    if rnd < 2:
      gather_role(d, 0, a_out)
      gather_role(d, 2, d_out)
    elif rnd < 4:
      gather_role(d, 2, c_out)
      gather_role(d, 0, f_out)
    elif rnd < 6:
      gather_role(d, 1, b_out)
    else:
      gather_role(d, 1, e_out)


def _phase2_tc(a_ref, b_ref, c_ref, d_ref, e_ref, f_ref, out_ref):
  prod = a_ref[...] * b_ref[...] * c_ref[...] \
      + d_ref[...] * e_ref[...] * f_ref[...]
  out_ref[...] = 0.5 * jnp.sum(prod, axis=0)


@jax.jit
def _simple_score(h_idx, r_idx, t_idx, headT, tailT, relT, rinvT):
  mesh = plsc.VectorSubcoreMesh(
      core_axis_name="c", subcore_axis_name="s",
      num_cores=_NC, num_subcores=_NS)
  gmat = jax.ShapeDtypeStruct((_D, _B), jnp.float32)
  params = pltpu.CompilerParams(needs_layout_passes=False)
  p1 = functools.partial(
      pl.kernel,
      out_type=(gmat,) * 6,
      mesh=mesh,
      compiler_params=params,
      scratch_types=[
          pltpu.VMEM((_E,), jnp.float32),
          pltpu.VMEM_SHARED((_B,), jnp.int32),
          pltpu.VMEM_SHARED((_B,), jnp.int32),
          pltpu.VMEM_SHARED((_B,), jnp.int32),
          pltpu.VMEM((_S,), jnp.int32),
          pltpu.VMEM((_S,), jnp.int32),
          pltpu.VMEM((_S,), jnp.float32),
          pltpu.VMEM((_S,), jnp.float32),
          pltpu.SemaphoreType.DMA,
          pltpu.SemaphoreType.DMA,
          pltpu.SemaphoreType.DMA,
          pltpu.SemaphoreType.DMA,
      ],
  )(_phase1_body)
  a, b, c, d, e, f = p1(headT, tailT, relT, rinvT, h_idx, r_idx, t_idx)

  blk = 2048
  in_spec = pl.BlockSpec((_D, blk), lambda i: (0, i))
  p2 = pl.pallas_call(
      _phase2_tc,
      out_shape=jax.ShapeDtypeStruct((_B,), jnp.float32),
      grid=(_B // blk,),
      in_specs=[in_spec] * 6,
      out_specs=pl.BlockSpec((blk,), lambda i: (i,)),
  )
  return p2(a, b, c, d, e, f)


def kernel(sample, head_embedding, tail_embedding, relation_embedding,
           relation_inverse_embedding):
  sample = sample.astype(jnp.int32)
  h_idx = sample[:, 0]
  r_idx = sample[:, 1]
  t_idx = sample[:, 2]
  return _simple_score(h_idx, r_idx, t_idx,
                       head_embedding.T, tail_embedding.T,
                       relation_embedding.T, relation_inverse_embedding.T)


# speedup vs baseline: 2.3315x; 1.0067x over previous
import functools

import jax
import jax.numpy as jnp
from jax import lax
from jax.experimental import pallas as pl
from jax.experimental.pallas import tpu as pltpu
from jax.experimental.pallas import tpu_sc as plsc

_B = 16384
_D = 64
_E = 100000
_L = 16
_NC = 2
_NS = 16
_NW = _NC * _NS
_PW = _B // _NW
_S = 4096
_NSTR = _B // _S


def _phase1_body(headT, tailT, relT, rinvT,
                 h_idx, r_idx, t_idx,
                 a_out, b_out, c_out, d_out, e_out, f_out,
                 plane_v, h_sp, r_sp, t_sp, idx0_v, idx1_v, val0_v, val1_v,
                 sem_i, sem_o):
  wid = lax.axis_index("s") * _NC + lax.axis_index("c")
  sid = lax.axis_index("s")
  idx_bufs = (idx0_v, idx1_v)
  val_bufs = (val0_v, val1_v)

  @pl.when(sid == 0)
  def _stage_idx():
    pltpu.sync_copy(h_idx, h_sp)
    pltpu.sync_copy(r_idx, r_sp)
    pltpu.sync_copy(t_idx, t_sp)
  plsc.subcore_barrier()
  idx_srcs = (h_sp, r_sp, t_sp)

  def gather_role(d, idx_row, out_hbm):
    idx_src = idx_srcs[idx_row]
    pltpu.async_copy(idx_src.at[pl.ds(0, _S)], idx_bufs[0], sem_i)
    out_cps = []
    for s in range(_NSTR):
      idx_v = idx_bufs[s % 2]
      val_v = val_bufs[s % 2]
      pltpu.make_async_copy(idx_src.at[pl.ds(s * _S, _S)], idx_v,
                            sem_i).wait()
      if s + 1 < _NSTR:
        pltpu.async_copy(idx_src.at[pl.ds((s + 1) * _S, _S)],
                         idx_bufs[(s + 1) % 2], sem_i)
      if s >= 2:
        out_cps[s - 2].wait()

      def gbody(g, carry):
        for u in range(16):
          sl = pl.ds((g * 16 + u) * _L, _L)
          val_v[sl] = plsc.load_gather(plane_v, [idx_v[sl]])
        return carry

      lax.fori_loop(0, _S // (16 * _L), gbody, 0)
      out_cps.append(
          pltpu.async_copy(val_v, out_hbm.at[d, pl.ds(s * _S, _S)], sem_o))
    for c in out_cps[max(0, _NSTR - 2):]:
      c.wait()

  for rnd in range(8):
    tbl = (headT, headT, tailT, tailT, relT, relT, rinvT, rinvT)[rnd]
    d = (rnd % 2) * 32 + wid
    pltpu.sync_copy(tbl.at[d], plane_v)
    if rnd < 2:
      gather_role(d, 0, a_out)
      gather_role(d, 2, d_out)
    elif rnd < 4:
      gather_role(d, 2, c_out)
      gather_role(d, 0, f_out)
    elif rnd < 6:
      gather_role(d, 1, b_out)
    else:
      gather_role(d, 1, e_out)


def _phase2_tc(a_ref, b_ref, c_ref, d_ref, e_ref, f_ref, out_ref):
  prod = a_ref[...] * b_ref[...] * c_ref[...] \
      + d_ref[...] * e_ref[...] * f_ref[...]
  out_ref[...] = 0.5 * jnp.sum(prod, axis=0)


@jax.jit
def _simple_score(h_idx, r_idx, t_idx, headT, tailT, relT, rinvT):
  mesh = plsc.VectorSubcoreMesh(
      core_axis_name="c", subcore_axis_name="s",
      num_cores=_NC, num_subcores=_NS)
  gmat = jax.ShapeDtypeStruct((_D, _B), jnp.float32)
  params = pltpu.CompilerParams(needs_layout_passes=False)
  p1 = functools.partial(
      pl.kernel,
      out_type=(gmat,) * 6,
      mesh=mesh,
      compiler_params=params,
      scratch_types=[
          pltpu.VMEM((_E,), jnp.float32),
          pltpu.VMEM_SHARED((_B,), jnp.int32),
          pltpu.VMEM_SHARED((_B,), jnp.int32),
          pltpu.VMEM_SHARED((_B,), jnp.int32),
          pltpu.VMEM((_S,), jnp.int32),
          pltpu.VMEM((_S,), jnp.int32),
          pltpu.VMEM((_S,), jnp.float32),
          pltpu.VMEM((_S,), jnp.float32),
          pltpu.SemaphoreType.DMA,
          pltpu.SemaphoreType.DMA,
      ],
  )(_phase1_body)
  a, b, c, d, e, f = p1(headT, tailT, relT, rinvT, h_idx, r_idx, t_idx)

  blk = 4096
  in_spec = pl.BlockSpec((_D, blk), lambda i: (0, i))
  p2 = pl.pallas_call(
      _phase2_tc,
      out_shape=jax.ShapeDtypeStruct((_B,), jnp.float32),
      grid=(_B // blk,),
      in_specs=[in_spec] * 6,
      out_specs=pl.BlockSpec((blk,), lambda i: (i,)),
  )
  return p2(a, b, c, d, e, f)


def kernel(sample, head_embedding, tail_embedding, relation_embedding,
           relation_inverse_embedding):
  sample = sample.astype(jnp.int32)
  h_idx = sample[:, 0]
  r_idx = sample[:, 1]
  t_idx = sample[:, 2]
  return _simple_score(h_idx, r_idx, t_idx,
                       head_embedding.T, tail_embedding.T,
                       relation_embedding.T, relation_inverse_embedding.T)


# speedup vs baseline: 2.3635x; 1.0137x over previous
import functools

import jax
import jax.numpy as jnp
from jax import lax
from jax.experimental import pallas as pl
from jax.experimental.pallas import tpu as pltpu
from jax.experimental.pallas import tpu_sc as plsc

_B = 16384
_D = 64
_E = 100000
_L = 16
_NC = 2
_NS = 16
_NW = _NC * _NS
_PW = _B // _NW
_S = 4096
_NSTR = _B // _S


def _phase1_body(headT, tailT, relT, rinvT,
                 h_idx, r_idx, t_idx,
                 a_out, b_out, c_out, d_out, e_out, f_out,
                 plane_v, h_sp, r_sp, t_sp, idx0_v, idx1_v, val0_v, val1_v,
                 sem_i, sem_o):
  wid = lax.axis_index("s") * _NC + lax.axis_index("c")
  sid = lax.axis_index("s")
  idx_bufs = (idx0_v, idx1_v)
  val_bufs = (val0_v, val1_v)

  @pl.when(sid == 0)
  def _stage_idx():
    pltpu.sync_copy(h_idx, h_sp)
    pltpu.sync_copy(r_idx, r_sp)
    pltpu.sync_copy(t_idx, t_sp)
  plsc.subcore_barrier()
  idx_srcs = (h_sp, r_sp, t_sp)

  def gather_role(d, idx_row, out_hbm, prefetched=False):
    idx_src = idx_srcs[idx_row]
    if not prefetched:
      pltpu.async_copy(idx_src.at[pl.ds(0, _S)], idx_bufs[0], sem_i)
    out_cps = []
    for s in range(_NSTR):
      idx_v = idx_bufs[s % 2]
      val_v = val_bufs[s % 2]
      pltpu.make_async_copy(idx_src.at[pl.ds(s * _S, _S)], idx_v,
                            sem_i).wait()
      if s + 1 < _NSTR:
        pltpu.async_copy(idx_src.at[pl.ds((s + 1) * _S, _S)],
                         idx_bufs[(s + 1) % 2], sem_i)
      if s >= 2:
        out_cps[s - 2].wait()

      def gbody(g, carry):
        for u in range(16):
          sl = pl.ds((g * 16 + u) * _L, _L)
          val_v[sl] = plsc.load_gather(plane_v, [idx_v[sl]])
        return carry

      lax.fori_loop(0, _S // (16 * _L), gbody, 0)
      out_cps.append(
          pltpu.async_copy(val_v, out_hbm.at[d, pl.ds(s * _S, _S)], sem_o))
    for c in out_cps[max(0, _NSTR - 2):]:
      c.wait()

  first_role = (0, 0, 2, 2, 1, 1, 1, 1)
  for rnd in range(8):
    tbl = (headT, headT, tailT, tailT, relT, relT, rinvT, rinvT)[rnd]
    d = (rnd % 2) * 32 + wid
    pltpu.async_copy(idx_srcs[first_role[rnd]].at[pl.ds(0, _S)],
                     idx_bufs[0], sem_i)
    pltpu.sync_copy(tbl.at[d], plane_v)
    if rnd < 2:
      gather_role(d, 0, a_out, prefetched=True)
      gather_role(d, 2, d_out)
    elif rnd < 4:
      gather_role(d, 2, c_out, prefetched=True)
      gather_role(d, 0, f_out)
    elif rnd < 6:
      gather_role(d, 1, b_out, prefetched=True)
    else:
      gather_role(d, 1, e_out, prefetched=True)


def _phase2_tc(a_ref, b_ref, c_ref, d_ref, e_ref, f_ref, out_ref):
  prod = a_ref[...] * b_ref[...] * c_ref[...] \
      + d_ref[...] * e_ref[...] * f_ref[...]
  out_ref[...] = 0.5 * jnp.sum(prod, axis=0)


@jax.jit
def _simple_score(h_idx, r_idx, t_idx, headT, tailT, relT, rinvT):
  mesh = plsc.VectorSubcoreMesh(
      core_axis_name="c", subcore_axis_name="s",
      num_cores=_NC, num_subcores=_NS)
  gmat = jax.ShapeDtypeStruct((_D, _B), jnp.float32)
  params = pltpu.CompilerParams(needs_layout_passes=False)
  p1 = functools.partial(
      pl.kernel,
      out_type=(gmat,) * 6,
      mesh=mesh,
      compiler_params=params,
      scratch_types=[
          pltpu.VMEM((_E,), jnp.float32),
          pltpu.VMEM_SHARED((_B,), jnp.int32),
          pltpu.VMEM_SHARED((_B,), jnp.int32),
          pltpu.VMEM_SHARED((_B,), jnp.int32),
          pltpu.VMEM((_S,), jnp.int32),
          pltpu.VMEM((_S,), jnp.int32),
          pltpu.VMEM((_S,), jnp.float32),
          pltpu.VMEM((_S,), jnp.float32),
          pltpu.SemaphoreType.DMA,
          pltpu.SemaphoreType.DMA,
      ],
  )(_phase1_body)
  a, b, c, d, e, f = p1(headT, tailT, relT, rinvT, h_idx, r_idx, t_idx)

  blk = 4096
  in_spec = pl.BlockSpec((_D, blk), lambda i: (0, i))
  p2 = pl.pallas_call(
      _phase2_tc,
      out_shape=jax.ShapeDtypeStruct((_B,), jnp.float32),
      grid=(_B // blk,),
      in_specs=[in_spec] * 6,
      out_specs=pl.BlockSpec((blk,), lambda i: (i,)),
  )
  return p2(a, b, c, d, e, f)


def kernel(sample, head_embedding, tail_embedding, relation_embedding,
           relation_inverse_embedding):
  sample = sample.astype(jnp.int32)
  h_idx = sample[:, 0]
  r_idx = sample[:, 1]
  t_idx = sample[:, 2]
  return _simple_score(h_idx, r_idx, t_idx,
                       head_embedding.T, tail_embedding.T,
                       relation_embedding.T, relation_inverse_embedding.T)
